# Initial kernel scaffold; baseline (speedup 1.0000x reference)
import jax
import jax.numpy as jnp
from jax.experimental import pallas as pl


def kernel(hidden_states, weight, e_score_correction_bias):
    raise NotImplementedError("write your pallas kernel here")


# speedup vs baseline: 1.3272x; 1.3272x over previous
import functools

import jax
import jax.numpy as jnp
from jax.experimental import pallas as pl

E = 64
TOP_K = 8
N_GROUP = 16
PER_GROUP = E // N_GROUP
TOPK_GROUP = 4
SCALE = 2.5

TB = 512


def _gate_kernel(x_ref, w_ref, b_ref, idx_ref, wt_ref):
    x = x_ref[...]
    w = w_ref[...]
    logits = jax.lax.dot_general(
        x, w, (((1,), (1,)), ((), ())), preferred_element_type=jnp.float32
    )
    scores = jax.nn.sigmoid(logits)
    sfc = scores + b_ref[...]

    tb = sfc.shape[0]
    grouped = sfc.reshape(tb, N_GROUP, PER_GROUP)
    g0 = grouped[:, :, 0]
    g1 = grouped[:, :, 1]
    g2 = grouped[:, :, 2]
    g3 = grouped[:, :, 3]
    gs = jnp.maximum(g0 + g1, g0 + g2)
    gs = jnp.maximum(gs, g0 + g3)
    gs = jnp.maximum(gs, g1 + g2)
    gs = jnp.maximum(gs, g1 + g3)
    gs = jnp.maximum(gs, g2 + g3)

    neg_inf = jnp.float32(-jnp.inf)
    cols64 = jax.lax.broadcasted_iota(jnp.int32, (tb, E), 1)
    col_group = cols64 // PER_GROUP

    mask64 = jnp.zeros((tb, E), dtype=jnp.bool_)
    gcur = gs
    for _ in range(TOPK_GROUP):
        gi = jnp.argmax(gcur, axis=1).astype(jnp.int32)
        mask64 = mask64 | (col_group == gi[:, None])
        gcur = jnp.where(
            jax.lax.broadcasted_iota(jnp.int32, gcur.shape, 1) == gi[:, None],
            neg_inf,
            gcur,
        )

    tmp = jnp.where(mask64, sfc, neg_inf)
    idx_cols = []
    wt_cols = []
    for _ in range(TOP_K):
        ii = jnp.argmax(tmp, axis=1).astype(jnp.int32)
        onehot = cols64 == ii[:, None]
        wv = jnp.sum(jnp.where(onehot, scores, 0.0), axis=1)
        tmp = jnp.where(onehot, neg_inf, tmp)
        idx_cols.append(ii[:, None])
        wt_cols.append(wv[:, None])

    idx = jnp.concatenate(idx_cols, axis=1)
    wt = jnp.concatenate(wt_cols, axis=1)
    denom = jnp.sum(wt, axis=1, keepdims=True) + 1e-20
    wt = wt / denom * SCALE

    idx_ref[...] = idx
    wt_ref[...] = wt


@functools.partial(jax.jit, static_argnames=())
def kernel(hidden_states, weight, e_score_correction_bias):
    bsz, seq_len, h = hidden_states.shape
    t = bsz * seq_len
    x = hidden_states.reshape(t, h).astype(jnp.float32)
    bias2d = e_score_correction_bias.reshape(1, E).astype(jnp.float32)
    grid = (t // TB,)
    idx, wt = pl.pallas_call(
        _gate_kernel,
        grid=grid,
        in_specs=[
            pl.BlockSpec((TB, h), lambda i: (i, 0)),
            pl.BlockSpec((E, h), lambda i: (0, 0)),
            pl.BlockSpec((1, E), lambda i: (0, 0)),
        ],
        out_specs=[
            pl.BlockSpec((TB, TOP_K), lambda i: (i, 0)),
            pl.BlockSpec((TB, TOP_K), lambda i: (i, 0)),
        ],
        out_shape=[
            jax.ShapeDtypeStruct((t, TOP_K), jnp.int32),
            jax.ShapeDtypeStruct((t, TOP_K), jnp.float32),
        ],
    )(x, weight.astype(jnp.float32), bias2d)
    return idx, wt


# speedup vs baseline: 3.4256x; 2.5811x over previous
import functools

import jax
import jax.numpy as jnp
from jax.experimental import pallas as pl

E = 64
TOP_K = 8
N_GROUP = 16
PER_GROUP = E // N_GROUP
TOPK_GROUP = 4
SCALE = 2.5

TB = 512


def _gate_kernel(x_ref, w_ref, b_ref, idx_ref, wt_ref):
    x = x_ref[...]
    w = w_ref[...]
    logits = jax.lax.dot_general(
        x, w, (((1,), (1,)), ((), ())), preferred_element_type=jnp.float32
    )
    scores = jax.nn.sigmoid(logits)
    sfc = scores + b_ref[...]

    tb = sfc.shape[0]
    neg_inf = jnp.float32(-jnp.inf)
    cols64 = jax.lax.broadcasted_iota(jnp.int32, (tb, E), 1)
    cmod = cols64 & (PER_GROUP - 1)
    col_group = cols64 // PER_GROUP

    s1 = jnp.roll(sfc, -1, axis=1)
    s2 = jnp.roll(sfc, -2, axis=1)
    s3 = jnp.roll(sfc, -3, axis=1)
    p = jnp.where(cmod <= 2, sfc + s1, neg_inf)
    p = jnp.maximum(p, jnp.where(cmod <= 1, sfc + s2, neg_inf))
    p = jnp.maximum(p, jnp.where(cmod == 0, sfc + s3, neg_inf))
    a = jnp.maximum(p, jnp.where(cmod <= 2, jnp.roll(p, -1, axis=1), neg_inf))
    b = jnp.maximum(a, jnp.where(cmod <= 1, jnp.roll(a, -2, axis=1), neg_inf))
    bhead = jnp.where(cmod == 0, b, neg_inf)

    mask64 = jnp.zeros((tb, E), dtype=jnp.bool_)
    for _ in range(TOPK_GROUP):
        gcol = jnp.argmax(bhead, axis=1).astype(jnp.int32)
        gsel = gcol[:, None]
        mask64 = mask64 | (col_group == (gsel // PER_GROUP))
        bhead = jnp.where(cols64 == gsel, neg_inf, bhead)

    tmp = jnp.where(mask64, sfc, neg_inf)
    idx_cols = []
    wt_cols = []
    for _ in range(TOP_K):
        ii = jnp.argmax(tmp, axis=1).astype(jnp.int32)
        onehot = cols64 == ii[:, None]
        wv = jnp.sum(jnp.where(onehot, scores, 0.0), axis=1)
        tmp = jnp.where(onehot, neg_inf, tmp)
        idx_cols.append(ii[:, None])
        wt_cols.append(wv[:, None])

    idx = jnp.concatenate(idx_cols, axis=1)
    wt = jnp.concatenate(wt_cols, axis=1)
    denom = jnp.sum(wt, axis=1, keepdims=True) + 1e-20
    wt = wt / denom * SCALE

    idx_ref[...] = idx
    wt_ref[...] = wt


@functools.partial(jax.jit, static_argnames=())
def kernel(hidden_states, weight, e_score_correction_bias):
    bsz, seq_len, h = hidden_states.shape
    t = bsz * seq_len
    x = hidden_states.reshape(t, h).astype(jnp.float32)
    bias2d = e_score_correction_bias.reshape(1, E).astype(jnp.float32)
    grid = (t // TB,)
    idx, wt = pl.pallas_call(
        _gate_kernel,
        grid=grid,
        in_specs=[
            pl.BlockSpec((TB, h), lambda i: (i, 0)),
            pl.BlockSpec((E, h), lambda i: (0, 0)),
            pl.BlockSpec((1, E), lambda i: (0, 0)),
        ],
        out_specs=[
            pl.BlockSpec((TB, TOP_K), lambda i: (i, 0)),
            pl.BlockSpec((TB, TOP_K), lambda i: (i, 0)),
        ],
        out_shape=[
            jax.ShapeDtypeStruct((t, TOP_K), jnp.int32),
            jax.ShapeDtypeStruct((t, TOP_K), jnp.float32),
        ],
    )(x, weight.astype(jnp.float32), bias2d)
    return idx, wt


# speedup vs baseline: 3.7682x; 1.1000x over previous
import functools

import jax
import jax.numpy as jnp
from jax.experimental import pallas as pl

E = 64
TOP_K = 8
N_GROUP = 16
PER_GROUP = E // N_GROUP
TOPK_GROUP = 4
SCALE = 2.5

TB = 512


def _gate_kernel(x_ref, w_ref, b_ref, idx_ref, wt_ref):
    x = x_ref[...]
    w = w_ref[...]
    logits = jax.lax.dot_general(
        x, w, (((1,), (1,)), ((), ())), preferred_element_type=jnp.float32
    )
    scores = jax.nn.sigmoid(logits)
    sfc = scores + b_ref[...]

    tb = sfc.shape[0]
    neg_inf = jnp.float32(-jnp.inf)
    cols64 = jax.lax.broadcasted_iota(jnp.int32, (tb, E), 1)
    cmod = cols64 & (PER_GROUP - 1)
    col_group = cols64 // PER_GROUP

    s1 = jnp.roll(sfc, -1, axis=1)
    s2 = jnp.roll(sfc, -2, axis=1)
    s3 = jnp.roll(sfc, -3, axis=1)
    p = jnp.where(cmod <= 2, sfc + s1, neg_inf)
    p = jnp.maximum(p, jnp.where(cmod <= 1, sfc + s2, neg_inf))
    p = jnp.maximum(p, jnp.where(cmod == 0, sfc + s3, neg_inf))
    a = jnp.maximum(p, jnp.where(cmod <= 2, jnp.roll(p, -1, axis=1), neg_inf))
    b = jnp.maximum(a, jnp.where(cmod <= 1, jnp.roll(a, -2, axis=1), neg_inf))
    bhead = jnp.where(cmod == 0, b, neg_inf)

    headcol = cols64 - cmod

    mask64 = jnp.zeros((tb, E), dtype=jnp.bool_)
    for _ in range(TOPK_GROUP):
        gsel = jnp.argmax(bhead, axis=1).astype(jnp.int32)[:, None]
        mask64 = mask64 | (headcol == gsel)
        bhead = jnp.where(cols64 == gsel, neg_inf, bhead)

    tmp = jnp.where(mask64, sfc, neg_inf)
    idx_cols = []
    wt_cols = []
    for _ in range(TOP_K):
        ii = jnp.argmax(tmp, axis=1).astype(jnp.int32)[:, None]
        onehot = cols64 == ii
        wv = jnp.sum(jnp.where(onehot, scores, 0.0), axis=1)
        tmp = jnp.where(onehot, neg_inf, tmp)
        idx_cols.append(ii)
        wt_cols.append(wv[:, None])

    idx = jnp.concatenate(idx_cols, axis=1)
    wt = jnp.concatenate(wt_cols, axis=1)
    denom = jnp.sum(wt, axis=1, keepdims=True) + 1e-20
    wt = wt / denom * SCALE

    idx_ref[...] = idx
    wt_ref[...] = wt


@functools.partial(jax.jit, static_argnames=())
def kernel(hidden_states, weight, e_score_correction_bias):
    bsz, seq_len, h = hidden_states.shape
    t = bsz * seq_len
    x = hidden_states.reshape(t, h).astype(jnp.float32)
    bias2d = e_score_correction_bias.reshape(1, E).astype(jnp.float32)
    grid = (t // TB,)
    idx, wt = pl.pallas_call(
        _gate_kernel,
        grid=grid,
        in_specs=[
            pl.BlockSpec((TB, h), lambda i: (i, 0)),
            pl.BlockSpec((E, h), lambda i: (0, 0)),
            pl.BlockSpec((1, E), lambda i: (0, 0)),
        ],
        out_specs=[
            pl.BlockSpec((TB, TOP_K), lambda i: (i, 0)),
            pl.BlockSpec((TB, TOP_K), lambda i: (i, 0)),
        ],
        out_shape=[
            jax.ShapeDtypeStruct((t, TOP_K), jnp.int32),
            jax.ShapeDtypeStruct((t, TOP_K), jnp.float32),
        ],
    )(x, weight.astype(jnp.float32), bias2d)
    return idx, wt


# speedup vs baseline: 5.3494x; 1.4196x over previous
import functools

import jax
import jax.numpy as jnp
from jax.experimental import pallas as pl

E = 64
TOP_K = 8
N_GROUP = 16
PER_GROUP = E // N_GROUP
TOPK_GROUP = 4
SCALE = 2.5

TB = 512


def _gate_kernel(x_ref, w_ref, b_ref, idx_ref, wt_ref):
    x = x_ref[...]
    w = w_ref[...]
    logits = jax.lax.dot_general(
        w, x, (((1,), (1,)), ((), ())), preferred_element_type=jnp.float32
    )
    scores = jax.nn.sigmoid(logits)
    sfc = scores + b_ref[...]

    tb = sfc.shape[1]
    neg_inf = jnp.float32(-jnp.inf)
    rows64 = jax.lax.broadcasted_iota(jnp.int32, (E, tb), 0)
    rmod = rows64 & (PER_GROUP - 1)
    headrow = rows64 - rmod

    s1 = jnp.roll(sfc, -1, axis=0)
    s2 = jnp.roll(sfc, -2, axis=0)
    s3 = jnp.roll(sfc, -3, axis=0)
    p = jnp.where(rmod <= 2, sfc + s1, neg_inf)
    p = jnp.maximum(p, jnp.where(rmod <= 1, sfc + s2, neg_inf))
    p = jnp.maximum(p, jnp.where(rmod == 0, sfc + s3, neg_inf))
    a = jnp.maximum(p, jnp.where(rmod <= 2, jnp.roll(p, -1, axis=0), neg_inf))
    b = jnp.maximum(a, jnp.where(rmod <= 1, jnp.roll(a, -2, axis=0), neg_inf))
    bhead = jnp.where(rmod == 0, b, neg_inf)

    mask64 = jnp.zeros((E, tb), dtype=jnp.bool_)
    for _ in range(TOPK_GROUP):
        gsel = jnp.argmax(bhead, axis=0).astype(jnp.int32)[None, :]
        mask64 = mask64 | (headrow == gsel)
        bhead = jnp.where(rows64 == gsel, neg_inf, bhead)

    tmp = jnp.where(mask64, sfc, neg_inf)
    idx_rows = []
    wt_rows = []
    for _ in range(TOP_K):
        ii = jnp.argmax(tmp, axis=0).astype(jnp.int32)[None, :]
        onehot = rows64 == ii
        wv = jnp.sum(jnp.where(onehot, scores, 0.0), axis=0)
        tmp = jnp.where(onehot, neg_inf, tmp)
        idx_rows.append(ii)
        wt_rows.append(wv[None, :])

    idx = jnp.concatenate(idx_rows, axis=0)
    wt = jnp.concatenate(wt_rows, axis=0)
    denom = jnp.sum(wt, axis=0, keepdims=True) + 1e-20
    wt = wt / denom * SCALE

    idx_ref[...] = idx
    wt_ref[...] = wt


@functools.partial(jax.jit, static_argnames=())
def kernel(hidden_states, weight, e_score_correction_bias):
    bsz, seq_len, h = hidden_states.shape
    t = bsz * seq_len
    x = hidden_states.reshape(t, h).astype(jnp.float32)
    bias2d = e_score_correction_bias.reshape(E, 1).astype(jnp.float32)
    grid = (t // TB,)
    idx_t, wt_t = pl.pallas_call(
        _gate_kernel,
        grid=grid,
        in_specs=[
            pl.BlockSpec((TB, h), lambda i: (i, 0)),
            pl.BlockSpec((E, h), lambda i: (0, 0)),
            pl.BlockSpec((E, 1), lambda i: (0, 0)),
        ],
        out_specs=[
            pl.BlockSpec((TOP_K, TB), lambda i: (0, i)),
            pl.BlockSpec((TOP_K, TB), lambda i: (0, i)),
        ],
        out_shape=[
            jax.ShapeDtypeStruct((TOP_K, t), jnp.int32),
            jax.ShapeDtypeStruct((TOP_K, t), jnp.float32),
        ],
    )(x, weight.astype(jnp.float32), bias2d)
    return idx_t.T, wt_t.T


# speedup vs baseline: 5.4459x; 1.0180x over previous
import functools

import jax
import jax.numpy as jnp
from jax.experimental import pallas as pl

E = 64
TOP_K = 8
N_GROUP = 16
PER_GROUP = E // N_GROUP
TOPK_GROUP = 4
SCALE = 2.5

TB = 512


def _gate_kernel(x_ref, w_ref, b_ref, idx_ref, wt_ref):
    x = x_ref[...]
    w = w_ref[...]
    logits_tok = jax.lax.dot_general(
        x, w, (((1,), (1,)), ((), ())), preferred_element_type=jnp.float32
    )
    logits = logits_tok.T
    scores = jax.nn.sigmoid(logits)
    sfc = scores + b_ref[...]

    tb = sfc.shape[1]
    neg_inf = jnp.float32(-jnp.inf)
    rows64 = jax.lax.broadcasted_iota(jnp.int32, (E, tb), 0)
    rmod = rows64 & (PER_GROUP - 1)
    headrow = rows64 - rmod

    s1 = jnp.roll(sfc, -1, axis=0)
    s2 = jnp.roll(sfc, -2, axis=0)
    s3 = jnp.roll(sfc, -3, axis=0)
    p = jnp.where(rmod <= 2, sfc + s1, neg_inf)
    p = jnp.maximum(p, jnp.where(rmod <= 1, sfc + s2, neg_inf))
    p = jnp.maximum(p, jnp.where(rmod == 0, sfc + s3, neg_inf))
    a = jnp.maximum(p, jnp.where(rmod <= 2, jnp.roll(p, -1, axis=0), neg_inf))
    b = jnp.maximum(a, jnp.where(rmod <= 1, jnp.roll(a, -2, axis=0), neg_inf))
    bhead = jnp.where(rmod == 0, b, neg_inf)

    mask64 = jnp.zeros((E, tb), dtype=jnp.bool_)
    for _ in range(TOPK_GROUP):
        gsel = jnp.argmax(bhead, axis=0).astype(jnp.int32)[None, :]
        mask64 = mask64 | (headrow == gsel)
        bhead = jnp.where(rows64 == gsel, neg_inf, bhead)

    tmp = jnp.where(mask64, sfc, neg_inf)
    idx_rows = []
    wt_rows = []
    for _ in range(TOP_K):
        ii = jnp.argmax(tmp, axis=0).astype(jnp.int32)[None, :]
        onehot = rows64 == ii
        wv = jnp.sum(jnp.where(onehot, scores, 0.0), axis=0)
        tmp = jnp.where(onehot, neg_inf, tmp)
        idx_rows.append(ii)
        wt_rows.append(wv[None, :])

    idx = jnp.concatenate(idx_rows, axis=0)
    wt = jnp.concatenate(wt_rows, axis=0)
    denom = jnp.sum(wt, axis=0, keepdims=True) + 1e-20
    wt = wt / denom * SCALE

    idx_ref[...] = idx
    wt_ref[...] = wt


@functools.partial(jax.jit, static_argnames=())
def kernel(hidden_states, weight, e_score_correction_bias):
    bsz, seq_len, h = hidden_states.shape
    t = bsz * seq_len
    x = hidden_states.reshape(t, h).astype(jnp.float32)
    bias2d = e_score_correction_bias.reshape(E, 1).astype(jnp.float32)
    grid = (t // TB,)
    idx_t, wt_t = pl.pallas_call(
        _gate_kernel,
        grid=grid,
        in_specs=[
            pl.BlockSpec((TB, h), lambda i: (i, 0)),
            pl.BlockSpec((E, h), lambda i: (0, 0)),
            pl.BlockSpec((E, 1), lambda i: (0, 0)),
        ],
        out_specs=[
            pl.BlockSpec((TOP_K, TB), lambda i: (0, i)),
            pl.BlockSpec((TOP_K, TB), lambda i: (0, i)),
        ],
        out_shape=[
            jax.ShapeDtypeStruct((TOP_K, t), jnp.int32),
            jax.ShapeDtypeStruct((TOP_K, t), jnp.float32),
        ],
    )(x, weight.astype(jnp.float32), bias2d)
    return idx_t.T, wt_t.T


# speedup vs baseline: 5.5932x; 1.0271x over previous
import functools

import jax
import jax.numpy as jnp
from jax.experimental import pallas as pl

E = 64
TOP_K = 8
N_GROUP = 16
PER_GROUP = E // N_GROUP
TOPK_GROUP = 4
SCALE = 2.5

TB = 1024


def _gate_kernel(x_ref, w_ref, b_ref, idx_ref, wt_ref):
    x = x_ref[...]
    w = w_ref[...]
    logits_tok = jax.lax.dot_general(
        x, w, (((1,), (1,)), ((), ())), preferred_element_type=jnp.float32
    )
    logits = logits_tok.T
    scores = jax.nn.sigmoid(logits)
    sfc = scores + b_ref[...]

    tb = sfc.shape[1]
    neg_inf = jnp.float32(-jnp.inf)
    rows64 = jax.lax.broadcasted_iota(jnp.int32, (E, tb), 0)
    rmod = rows64 & (PER_GROUP - 1)
    headrow = rows64 - rmod

    s1 = jnp.roll(sfc, -1, axis=0)
    s2 = jnp.roll(sfc, -2, axis=0)
    s3 = jnp.roll(sfc, -3, axis=0)
    p = jnp.where(rmod <= 2, sfc + s1, neg_inf)
    p = jnp.maximum(p, jnp.where(rmod <= 1, sfc + s2, neg_inf))
    p = jnp.maximum(p, jnp.where(rmod == 0, sfc + s3, neg_inf))
    a = jnp.maximum(p, jnp.where(rmod <= 2, jnp.roll(p, -1, axis=0), neg_inf))
    b = jnp.maximum(a, jnp.where(rmod <= 1, jnp.roll(a, -2, axis=0), neg_inf))
    bhead = jnp.where(rmod == 0, b, neg_inf)

    mask64 = jnp.zeros((E, tb), dtype=jnp.bool_)
    for _ in range(TOPK_GROUP):
        gsel = jnp.argmax(bhead, axis=0).astype(jnp.int32)[None, :]
        mask64 = mask64 | (headrow == gsel)
        bhead = jnp.where(rows64 == gsel, neg_inf, bhead)

    tmp = jnp.where(mask64, sfc, neg_inf)
    idx_rows = []
    wt_rows = []
    for _ in range(TOP_K):
        ii = jnp.argmax(tmp, axis=0).astype(jnp.int32)[None, :]
        onehot = rows64 == ii
        wv = jnp.sum(jnp.where(onehot, scores, 0.0), axis=0)
        tmp = jnp.where(onehot, neg_inf, tmp)
        idx_rows.append(ii)
        wt_rows.append(wv[None, :])

    idx = jnp.concatenate(idx_rows, axis=0)
    wt = jnp.concatenate(wt_rows, axis=0)
    denom = jnp.sum(wt, axis=0, keepdims=True) + 1e-20
    wt = wt / denom * SCALE

    idx_ref[...] = idx
    wt_ref[...] = wt


@functools.partial(jax.jit, static_argnames=())
def kernel(hidden_states, weight, e_score_correction_bias):
    bsz, seq_len, h = hidden_states.shape
    t = bsz * seq_len
    x = hidden_states.reshape(t, h).astype(jnp.float32)
    bias2d = e_score_correction_bias.reshape(E, 1).astype(jnp.float32)
    grid = (t // TB,)
    idx_t, wt_t = pl.pallas_call(
        _gate_kernel,
        grid=grid,
        in_specs=[
            pl.BlockSpec((TB, h), lambda i: (i, 0)),
            pl.BlockSpec((E, h), lambda i: (0, 0)),
            pl.BlockSpec((E, 1), lambda i: (0, 0)),
        ],
        out_specs=[
            pl.BlockSpec((TOP_K, TB), lambda i: (0, i)),
            pl.BlockSpec((TOP_K, TB), lambda i: (0, i)),
        ],
        out_shape=[
            jax.ShapeDtypeStruct((TOP_K, t), jnp.int32),
            jax.ShapeDtypeStruct((TOP_K, t), jnp.float32),
        ],
    )(x, weight.astype(jnp.float32), bias2d)
    return idx_t.T, wt_t.T
